# Initial kernel scaffold; baseline (speedup 1.0000x reference)
import jax
import jax.numpy as jnp
from jax.experimental import pallas as pl


def kernel(x, edge_index, batch, Wp, bp, W1, b1, W2, b2, Wf, bf):
    raise NotImplementedError("write your pallas kernel here")


# speedup vs baseline: 9.1346x; 9.1346x over previous
import functools

import jax
import jax.numpy as jnp
from jax import lax
from jax.experimental import pallas as pl
from jax.experimental.pallas import tpu as pltpu
from jax.experimental.pallas import tpu_sc as plsc

_N = 10000
_E = 320000
_F = 128
_G = 64

_NSC = 2
_NSUB = 16
_NTILES = _NSC * _NSUB
_EPT = _E // _NTILES
_CHUNK = 80
_NCHUNK = _EPT // _CHUNK
_RPT = 624
_RTAIL = _N - _NSUB * _RPT

_ROWS_BLK = 1000
_NBLK = _N // _ROWS_BLK

_mesh = plsc.VectorSubcoreMesh(core_axis_name="c", subcore_axis_name="s")


def _sc_agg_body(h_hbm, zeros_hbm, src_hbm, dst_hbm, out_hbm,
                 srcv, dstv, rows2, aggsh, gs0, gs1, ss0, ss1):
    cid = lax.axis_index("c")
    sid = lax.axis_index("s")
    tid = cid * _NSUB + sid
    rbase = sid * _RPT

    @pl.when(cid == 0)
    def _():
        pltpu.sync_copy(h_hbm.at[pl.ds(rbase, _RPT)],
                        aggsh.at[pl.ds(rbase, _RPT)])

        @pl.when(sid == _NSUB - 1)
        def _():
            pltpu.sync_copy(h_hbm.at[pl.ds(_NSUB * _RPT, _RTAIL)],
                            aggsh.at[pl.ds(_NSUB * _RPT, _RTAIL)])

    @pl.when(cid != 0)
    def _():
        pltpu.sync_copy(zeros_hbm, aggsh.at[pl.ds(rbase, _RPT)])

        @pl.when(sid == _NSUB - 1)
        def _():
            pltpu.sync_copy(zeros_hbm.at[pl.ds(0, _RTAIL)],
                            aggsh.at[pl.ds(_NSUB * _RPT, _RTAIL)])

    pltpu.sync_copy(src_hbm.at[pl.ds(tid * _EPT, _EPT)], srcv)
    pltpu.sync_copy(dst_hbm.at[tid], dstv)
    plsc.subcore_barrier()

    rA = rows2.at[pl.ds(0, _CHUNK)]
    rB = rows2.at[pl.ds(_CHUNK, _CHUNK)]

    def _gather(j, rbuf, sem):
        return pltpu.async_copy(h_hbm.at[srcv.at[pl.ds(j * _CHUNK, _CHUNK)]],
                                rbuf, sem)

    def _gwait(j, rbuf, sem):
        pltpu.make_async_copy(h_hbm.at[srcv.at[pl.ds(j * _CHUNK, _CHUNK)]],
                              rbuf, sem).wait()

    def _swait(j, rbuf, sem):
        pltpu.make_async_copy(rbuf, aggsh.at[dstv.at[j]], sem).wait()

    _gather(0, rA, gs0)

    def body(k, carry):
        j0 = 2 * k
        j1 = j0 + 1
        _gwait(j0, rA, gs0)

        @pl.when(k > 0)
        def _():
            _swait(j1 - 2, rB, ss1)

        _gather(j1, rB, gs1)
        pltpu.async_copy(rA, aggsh.at[dstv.at[j0]], ss0, add=True)
        _gwait(j1, rB, gs1)
        _swait(j0, rA, ss0)
        _gather(j0 + 2, rA, gs0)
        pltpu.async_copy(rB, aggsh.at[dstv.at[j1]], ss1, add=True)
        return carry

    lax.fori_loop(0, _NCHUNK // 2, body, 0)
    _gwait(_NCHUNK - 1, rA, gs0)
    _swait(_NCHUNK - 2, rB, ss1)
    pltpu.sync_copy(rA, aggsh.at[dstv.at[_NCHUNK - 1]], add=True)

    plsc.subcore_barrier()
    pltpu.sync_copy(aggsh.at[pl.ds(rbase, _RPT)],
                    out_hbm.at[pl.ds(cid * _N + rbase, _RPT)])

    @pl.when(sid == _NSUB - 1)
    def _():
        pltpu.sync_copy(aggsh.at[pl.ds(_NSUB * _RPT, _RTAIL)],
                        out_hbm.at[pl.ds(cid * _N + _NSUB * _RPT, _RTAIL)])


_sc_agg = functools.partial(
    pl.kernel,
    out_type=jax.ShapeDtypeStruct((_NSC * _N, _F), jnp.float32),
    mesh=_mesh,
    scratch_types=[
        pltpu.VMEM((_EPT,), jnp.int32),
        pltpu.VMEM((_NCHUNK, _CHUNK), jnp.int32),
        pltpu.VMEM((2 * _CHUNK, _F), jnp.float32),
        pltpu.VMEM_SHARED((_N, _F), jnp.float32),
        pltpu.SemaphoreType.DMA,
        pltpu.SemaphoreType.DMA,
        pltpu.SemaphoreType.DMA,
        pltpu.SemaphoreType.DMA,
    ],
)(_sc_agg_body)


def _dot_t(a, b):
    return lax.dot_general(a, b, (((1,), (1,)), ((), ())),
                           preferred_element_type=jnp.float32)


def _proj_body(x_ref, w_ref, b_ref, o_ref):
    o_ref[...] = _dot_t(x_ref[...], w_ref[...]) + b_ref[...]


def _proj(x, Wp, bp):
    return pl.pallas_call(
        _proj_body,
        grid=(_NBLK,),
        in_specs=[pl.BlockSpec((_ROWS_BLK, _F), lambda i: (i, 0)),
                  pl.BlockSpec((_F, _F), lambda i: (0, 0)),
                  pl.BlockSpec((1, _F), lambda i: (0, 0))],
        out_specs=pl.BlockSpec((_ROWS_BLK, _F), lambda i: (i, 0)),
        out_shape=jax.ShapeDtypeStruct((_N, _F), jnp.float32),
    )(x, Wp, bp)


def _mlp_body(p0_ref, p1_ref, w1_ref, b1_ref, w2_ref, b2_ref, o_ref):
    z = p0_ref[...] + p1_ref[...]
    z = jnp.maximum(_dot_t(z, w1_ref[...]) + b1_ref[...], 0.0)
    z = jnp.maximum(_dot_t(z, w2_ref[...]) + b2_ref[...], 0.0)
    o_ref[...] = z


def _mlp(parts, W1i, b1i, W2i, b2i):
    return pl.pallas_call(
        _mlp_body,
        grid=(_NBLK,),
        in_specs=[pl.BlockSpec((_ROWS_BLK, _F), lambda i: (i, 0)),
                  pl.BlockSpec((_ROWS_BLK, _F), lambda i: (i + _NBLK, 0)),
                  pl.BlockSpec((_F, _F), lambda i: (0, 0)),
                  pl.BlockSpec((1, _F), lambda i: (0, 0)),
                  pl.BlockSpec((_F, _F), lambda i: (0, 0)),
                  pl.BlockSpec((1, _F), lambda i: (0, 0))],
        out_specs=pl.BlockSpec((_ROWS_BLK, _F), lambda i: (i, 0)),
        out_shape=jax.ShapeDtypeStruct((_N, _F), jnp.float32),
    )(parts, parts, W1i, b1i, W2i, b2i)


def _pool_body(h_ref, b_ref, wf_ref, bf_ref, o_ref, sums, counts):
    i = pl.program_id(0)

    @pl.when(i == 0)
    def _():
        sums[...] = jnp.zeros_like(sums)
        counts[...] = jnp.zeros_like(counts)

    bvec = b_ref[0]
    gids = lax.broadcasted_iota(jnp.int32, (_G, _ROWS_BLK), 0)
    mask = (gids == bvec).astype(jnp.float32)
    sums[...] += lax.dot_general(mask, h_ref[...], (((1,), (0,)), ((), ())),
                                 preferred_element_type=jnp.float32)
    counts[...] += jnp.broadcast_to(
        jnp.sum(mask, axis=1, keepdims=True), (_G, _F))

    @pl.when(i == _NBLK - 1)
    def _():
        pooled = sums[...] / jnp.maximum(counts[...], 1.0)
        o_ref[...] = _dot_t(pooled, wf_ref[...]) + bf_ref[...]


def _pool(h, batch3, Wf, bf):
    return pl.pallas_call(
        _pool_body,
        grid=(_NBLK,),
        in_specs=[pl.BlockSpec((_ROWS_BLK, _F), lambda i: (i, 0)),
                  pl.BlockSpec((1, 1, _ROWS_BLK), lambda i: (i, 0, 0)),
                  pl.BlockSpec((_F, _F), lambda i: (0, 0)),
                  pl.BlockSpec((1, _F), lambda i: (0, 0))],
        out_specs=pl.BlockSpec((_G, _F), lambda i: (0, 0)),
        out_shape=jax.ShapeDtypeStruct((_G, _F), jnp.float32),
        scratch_shapes=[pltpu.VMEM((_G, _F), jnp.float32),
                        pltpu.VMEM((_G, _F), jnp.float32)],
    )(h, batch3, Wf, bf)


def kernel(x, edge_index, batch, Wp, bp, W1, b1, W2, b2, Wf, bf):
    src1 = edge_index[0]
    dst3 = edge_index[1].reshape(_NTILES, _NCHUNK, _CHUNK)
    zeros = jnp.zeros((_RPT, _F), jnp.float32)
    batch3 = batch.reshape(_NBLK, 1, _ROWS_BLK)

    h = _proj(x, Wp, bp.reshape(1, _F))
    for i in range(W1.shape[0]):
        parts = _sc_agg(h, zeros, src1, dst3)
        h = _mlp(parts, W1[i], b1[i].reshape(1, _F),
                 W2[i], b2[i].reshape(1, _F))
    return _pool(h, batch3, Wf, bf.reshape(1, _F))
